# baseline (speedup 1.0000x reference)
import jax
import jax.numpy as jnp
from jax.experimental import pallas as pl
from jax.experimental.pallas import tpu as pltpu

NE = 8
TOPK = 2
TB = 256
FB = 512


def _router_body(x_ref, gw_ref, logits_ref, cw_ref):
    x = x_ref[...]
    gw = gw_ref[...]
    logits = jax.lax.dot_general(x, gw, (((1,), (1,)), ((), ())),
                                 preferred_element_type=jnp.float32)
    logits_ref[...] = logits
    m = jnp.max(logits, axis=1, keepdims=True)
    e = jnp.exp(logits - m)
    p = e / jnp.sum(e, axis=1, keepdims=True)
    iota = jax.lax.broadcasted_iota(jnp.int32, p.shape, 1)
    v0 = jnp.max(p, axis=1, keepdims=True)
    i0 = jnp.min(jnp.where(p == v0, iota, NE), axis=1, keepdims=True)
    p2 = jnp.where(iota == i0, -1.0, p)
    v1 = jnp.max(p2, axis=1, keepdims=True)
    i1 = jnp.min(jnp.where(p2 == v1, iota, NE), axis=1, keepdims=True)
    s = v0 + v1
    cw = (jnp.where(iota == i0, v0 / s, 0.0)
          + jnp.where(iota == i1, v1 / s, 0.0))
    cw_ref[...] = cw


def _moe_body(cw_ref, x_ref, w1_ref, w3_ref, w2_ref, out_ref):
    e = pl.program_id(0)
    f = pl.program_id(1)
    t = pl.program_id(2)
    xb = x_ref[pl.ds(t * TB, TB), :]
    w1b = w1_ref[0]
    w3b = w3_ref[0]
    w2b = w2_ref[0]
    g = jax.lax.dot_general(xb, w1b, (((1,), (1,)), ((), ())),
                            preferred_element_type=jnp.float32)
    u = jax.lax.dot_general(xb, w3b, (((1,), (1,)), ((), ())),
                            preferred_element_type=jnp.float32)
    h = g * jax.lax.logistic(g) * u
    cwb = cw_ref[pl.ds(t * TB, TB), :]
    onehot = (jax.lax.broadcasted_iota(jnp.int32, (NE, 1), 0) == e
              ).astype(jnp.float32)
    wcol = jax.lax.dot_general(cwb, onehot, (((1,), (0,)), ((), ())),
                               preferred_element_type=jnp.float32)
    h = h * wcol
    y = jax.lax.dot_general(h, w2b, (((1,), (1,)), ((), ())),
                            preferred_element_type=jnp.float32)

    @pl.when(jnp.logical_and(e == 0, f == 0))
    def _init():
        out_ref[pl.ds(t * TB, TB), :] = y

    @pl.when(jnp.logical_not(jnp.logical_and(e == 0, f == 0)))
    def _acc():
        out_ref[pl.ds(t * TB, TB), :] += y


def kernel(hidden_states, gate_weight, w1, w3, w2):
    B, S, H = hidden_states.shape
    F = w1.shape[1]
    x = hidden_states.reshape(S, H)

    logits, cw = pl.pallas_call(
        _router_body,
        out_shape=[
            jax.ShapeDtypeStruct((S, NE), jnp.float32),
            jax.ShapeDtypeStruct((S, NE), jnp.float32),
        ],
    )(x, gate_weight)

    out = pl.pallas_call(
        _moe_body,
        grid=(NE, F // FB, S // TB),
        in_specs=[
            pl.BlockSpec((S, NE), lambda e, f, t: (0, 0)),
            pl.BlockSpec((S, H), lambda e, f, t: (0, 0)),
            pl.BlockSpec((1, FB, H), lambda e, f, t: (e, f, 0)),
            pl.BlockSpec((1, FB, H), lambda e, f, t: (e, f, 0)),
            pl.BlockSpec((1, H, FB), lambda e, f, t: (e, 0, f)),
        ],
        out_specs=pl.BlockSpec((S, H), lambda e, f, t: (0, 0)),
        out_shape=jax.ShapeDtypeStruct((S, H), jnp.float32),
        compiler_params=pltpu.CompilerParams(
            dimension_semantics=("arbitrary", "arbitrary", "arbitrary"),
        ),
    )(cw, x, w1, w3, w2)

    return (out.reshape(B, S, H), logits)


# speedup vs baseline: 1.8260x; 1.8260x over previous
import functools

import jax
import jax.numpy as jnp
from jax import lax
from jax.experimental import pallas as pl
from jax.experimental.pallas import tpu as pltpu
from jax.experimental.pallas import tpu_sc as plsc

NE = 8
TOPK = 2
BLK = 256
FB = 512
NB = 40
NW = 32
GCH = 80
CCH = 32


def _router_body(x_ref, gw_ref, logits_ref, sel_ref, wsel_ref):
    x = x_ref[...]
    gw = gw_ref[...]
    logits = lax.dot_general(x, gw, (((1,), (1,)), ((), ())),
                             preferred_element_type=jnp.float32)
    logits_ref[...] = logits
    m = jnp.max(logits, axis=1, keepdims=True)
    e = jnp.exp(logits - m)
    p = e / jnp.sum(e, axis=1, keepdims=True)
    iota = lax.broadcasted_iota(jnp.int32, p.shape, 1)
    v0 = jnp.max(p, axis=1, keepdims=True)
    i0 = jnp.min(jnp.where(p == v0, iota, NE), axis=1, keepdims=True)
    p2 = jnp.where(iota == i0, -1.0, p)
    v1 = jnp.max(p2, axis=1, keepdims=True)
    i1 = jnp.min(jnp.where(p2 == v1, iota, NE), axis=1, keepdims=True)
    s = v0 + v1
    sel_ref[...] = jnp.concatenate([i0, i1], axis=1)
    wsel_ref[...] = jnp.concatenate([v0 / s, v1 / s], axis=1)


def _gemm_body(m_ref, xs_ref, rw_ref, w1_ref, w3_ref, w2_ref, ys_ref):
    b = pl.program_id(0)
    f = pl.program_id(1)

    @pl.when(b < m_ref[NB])
    def _():
        xb = xs_ref[...]
        g = lax.dot_general(xb, w1_ref[0], (((1,), (1,)), ((), ())),
                            preferred_element_type=jnp.float32)
        u = lax.dot_general(xb, w3_ref[0], (((1,), (1,)), ((), ())),
                            preferred_element_type=jnp.float32)
        h = g * lax.logistic(g) * u * rw_ref[0]
        y = lax.dot_general(h, w2_ref[0], (((1,), (1,)), ((), ())),
                            preferred_element_type=jnp.float32)

        @pl.when(f == 0)
        def _init():
            ys_ref[...] = y

        @pl.when(f != 0)
        def _acc():
            ys_ref[...] += y


def _sc_gather(x_hbm, tok_hbm, out_hbm, idx_v, rows_v, sem):
    wid = lax.axis_index("s") * 2 + lax.axis_index("c")
    rows_per_w = out_hbm.shape[0] // NW
    base = wid * rows_per_w

    def body(i, carry):
        b = base + i * GCH
        pltpu.sync_copy(tok_hbm.at[pl.ds(b, GCH)], idx_v)
        pltpu.async_copy(x_hbm.at[idx_v], rows_v, sem).wait()
        pltpu.sync_copy(rows_v, out_hbm.at[pl.ds(b, GCH)])
        return carry

    lax.fori_loop(0, rows_per_w // GCH, body, 0)


def _sc_combine(ys_hbm, p0_hbm, p1_hbm, out_hbm, i0_v, i1_v, a_v, b_v, sem):
    wid = lax.axis_index("s") * 2 + lax.axis_index("c")
    toks_per_w = out_hbm.shape[0] // NW
    base = wid * toks_per_w
    ncol = out_hbm.shape[1] // 16

    def body(i, carry):
        b = base + i * CCH
        pltpu.sync_copy(p0_hbm.at[pl.ds(b, CCH)], i0_v)
        pltpu.sync_copy(p1_hbm.at[pl.ds(b, CCH)], i1_v)
        pltpu.async_copy(ys_hbm.at[i0_v], a_v, sem).wait()
        pltpu.async_copy(ys_hbm.at[i1_v], b_v, sem).wait()

        def row_add(r, c):
            for j in range(ncol):
                a_v[r, pl.ds(j * 16, 16)] = (a_v[r, pl.ds(j * 16, 16)]
                                             + b_v[r, pl.ds(j * 16, 16)])
            return c

        lax.fori_loop(0, CCH, row_add, 0)
        pltpu.sync_copy(a_v, out_hbm.at[pl.ds(b, CCH)])
        return carry

    lax.fori_loop(0, toks_per_w // CCH, body, 0)


def kernel(hidden_states, gate_weight, w1, w3, w2):
    B, S, H = hidden_states.shape
    F = w1.shape[1]
    A = TOPK * S
    RP = NB * BLK
    x = hidden_states.reshape(S, H)

    logits, sel, wsel = pl.pallas_call(
        _router_body,
        out_shape=[
            jax.ShapeDtypeStruct((S, NE), jnp.float32),
            jax.ShapeDtypeStruct((S, TOPK), jnp.int32),
            jax.ShapeDtypeStruct((S, TOPK), jnp.float32),
        ],
    )(x, gate_weight)

    e_flat = sel.T.reshape(A)
    w_flat = wsel.T.reshape(A)
    t_flat = jnp.concatenate([jnp.arange(S, dtype=jnp.int32)] * TOPK)
    oh = (e_flat[:, None] == jnp.arange(NE)[None, :]).astype(jnp.int32)
    csum = jnp.cumsum(oh, axis=0)
    rank = jnp.sum((csum - oh) * oh, axis=1)
    counts = csum[-1]
    padded = ((counts + BLK - 1) // BLK) * BLK
    off_end = jnp.cumsum(padded)
    off = off_end - padded
    pos = (jnp.sum(oh * off[None, :], axis=1) + rank).astype(jnp.int32)
    row_token = jnp.zeros((RP,), jnp.int32).at[pos].set(t_flat)
    row_w = jnp.zeros((RP,), jnp.float32).at[pos].set(w_flat)
    n_used = (off_end[-1] // BLK).astype(jnp.int32)
    bidx = jnp.arange(NB, dtype=jnp.int32)
    be_raw = jnp.minimum(
        jnp.sum((bidx[:, None] * BLK >= off_end[None, :]).astype(jnp.int32),
                axis=1), NE - 1).astype(jnp.int32)
    be = jnp.where(bidx >= n_used, be_raw[n_used - 1], be_raw)
    meta = jnp.concatenate([be, n_used[None]])
    p0 = pos[:S]
    p1 = pos[S:]

    xs = functools.partial(
        pl.kernel,
        mesh=plsc.VectorSubcoreMesh(core_axis_name="c", subcore_axis_name="s"),
        out_type=jax.ShapeDtypeStruct((RP, H), jnp.float32),
        scratch_types=[
            pltpu.VMEM((GCH,), jnp.int32),
            pltpu.VMEM((GCH, H), jnp.float32),
            pltpu.SemaphoreType.DMA,
        ],
    )(_sc_gather)(x, row_token)

    def _w13_map(b, f, m):
        dead = b >= m[NB]
        return (m[b], jnp.where(dead, F // FB - 1, f), 0)

    def _w2_map(b, f, m):
        dead = b >= m[NB]
        return (m[b], 0, jnp.where(dead, F // FB - 1, f))

    ys = pl.pallas_call(
        _gemm_body,
        grid_spec=pltpu.PrefetchScalarGridSpec(
            num_scalar_prefetch=1,
            grid=(NB, F // FB),
            in_specs=[
                pl.BlockSpec((BLK, H), lambda b, f, m: (b, 0)),
                pl.BlockSpec((1, BLK, 1), lambda b, f, m: (b, 0, 0)),
                pl.BlockSpec((1, FB, H), _w13_map),
                pl.BlockSpec((1, FB, H), _w13_map),
                pl.BlockSpec((1, H, FB), _w2_map),
            ],
            out_specs=pl.BlockSpec((BLK, H), lambda b, f, m: (b, 0)),
        ),
        out_shape=jax.ShapeDtypeStruct((RP, H), jnp.float32),
        compiler_params=pltpu.CompilerParams(
            dimension_semantics=("arbitrary", "arbitrary"),
        ),
    )(meta, xs, row_w.reshape(NB, BLK, 1), w1, w3, w2)

    out = functools.partial(
        pl.kernel,
        mesh=plsc.VectorSubcoreMesh(core_axis_name="c", subcore_axis_name="s"),
        out_type=jax.ShapeDtypeStruct((S, H), jnp.float32),
        scratch_types=[
            pltpu.VMEM((CCH,), jnp.int32),
            pltpu.VMEM((CCH,), jnp.int32),
            pltpu.VMEM((CCH, H), jnp.float32),
            pltpu.VMEM((CCH, H), jnp.float32),
            pltpu.SemaphoreType.DMA,
        ],
    )(_sc_combine)(ys, p0, p1)

    return (out.reshape(B, S, H), logits)
